# baseline (speedup 1.0000x reference)
import functools

import jax
import jax.numpy as jnp
from jax import lax
from jax.experimental import pallas as pl
from jax.experimental.pallas import tpu as pltpu
from jax.experimental.pallas import tpu_sc as plsc

_B = 16
_SEQ = 2048
_ENC = 1024
_T = 50

_NC = 2
_NS = 16
_NW = _NC * _NS
_LANES = 16
_PAIRS = _B * _T
_ROWS_PER_W = _PAIRS // _NW
_IDX_W = 32
_CHUNKS = _ENC // _LANES


def _sc_body(table_hbm, ids_hbm, conw_hbm, e_hbm, ids_v, idx_a, idx_b,
             rows_a, rows_b, w_v, e_v, sem, sem_b, sem_w):
    wid = lax.axis_index("s") * _NC + lax.axis_index("c")
    b = wid // 2
    half = wid % 2
    cp_w = pltpu.async_copy(conw_hbm.at[0, pl.ds(_ENC, _ENC)], w_v, sem_w)
    base = wid * _ROWS_PER_W
    abase = (base // 8) * 8
    sh = base - abase
    pltpu.sync_copy(ids_hbm.at[pl.ds(abase, 32)], ids_v.at[pl.ds(0, 32)])
    lane = lax.iota(jnp.int32, _LANES)
    off = b * _SEQ
    nb = _ROWS_PER_W - _LANES
    idx_a[pl.ds(0, _LANES)] = ids_v[pl.ds(sh, _LANES)] + off
    tail = ids_v[pl.ds(sh + _LANES, _LANES)] + off
    npad = _IDX_W - _ROWS_PER_W
    pad_vals = wid * npad + (lane - nb)
    idx_b[pl.ds(0, _LANES)] = jnp.where(lane < nb, tail, pad_vals)
    cp_a = pltpu.async_copy(table_hbm.at[idx_a], rows_a, sem)
    cp_b = pltpu.async_copy(table_hbm.at[idx_b], rows_b, sem_b)
    cp_w.wait()

    def make_mac(rows_ref, nrows):
        def mac(l, accs):
            off = l * _LANES
            w_c = w_v[pl.ds(off, _LANES)]
            return tuple(accs[r] + rows_ref[r, pl.ds(off, _LANES)] * w_c
                         for r in range(nrows))
        return mac
    zero = jnp.zeros((_LANES,), jnp.float32)
    cp_a.wait()
    accs_a = lax.fori_loop(0, _CHUNKS, make_mac(rows_a, _LANES),
                           (zero,) * _LANES)
    cp_b.wait()
    nb = _ROWS_PER_W - _LANES
    accs_b = lax.fori_loop(0, _CHUNKS, make_mac(rows_b, nb), (zero,) * nb)
    for r, acc in enumerate(accs_a + accs_b):
        e_v[pl.ds(r * _LANES, _LANES)] = acc
    for r in range(_ROWS_PER_W, _IDX_W):
        e_v[pl.ds(r * _LANES, _LANES)] = zero
    pltpu.sync_copy(e_v, e_hbm.at[b, pl.ds(half * (_IDX_W * _LANES),
                                           _IDX_W * _LANES)])


@functools.lru_cache(maxsize=1)
def _sc_gather_dot():
    return functools.partial(
        pl.kernel,
        mesh=plsc.VectorSubcoreMesh(core_axis_name="c", subcore_axis_name="s"),
        out_type=jax.ShapeDtypeStruct((_B, 2 * _IDX_W * _LANES), jnp.float32),
        scratch_types=[
            pltpu.VMEM((48,), jnp.int32),
            pltpu.VMEM((_LANES,), jnp.int32),
            pltpu.VMEM((_LANES,), jnp.int32),
            pltpu.VMEM((_LANES, _ENC), jnp.float32),
            pltpu.VMEM((_LANES, _ENC), jnp.float32),
            pltpu.VMEM((_ENC,), jnp.float32),
            pltpu.VMEM((_IDX_W * _LANES,), jnp.float32),
            pltpu.SemaphoreType.DMA,
            pltpu.SemaphoreType.DMA,
            pltpu.SemaphoreType.DMA,
        ],
    )(_sc_body)


def _loss_body(x_ref, out_ref):
    x = x_ref[:, :]
    _W = _IDX_W * _LANES
    ii = lax.broadcasted_iota(jnp.int32, (2 * _W, _T), 0)
    tt = lax.broadcasted_iota(jnp.int32, (2 * _W, _T), 1)
    slot = (ii % _W) // _LANES
    half = ii // _W
    fold = jnp.where((slot < _ROWS_PER_W) & (half * _ROWS_PER_W + slot == tt),
                     jnp.float32(1.0), jnp.float32(0.0))
    e = jnp.dot(x, fold, preferred_element_type=jnp.float32)
    kcol = lax.broadcasted_iota(jnp.int32, (_B, _T), 1)
    valid = kcol >= 1
    m = jnp.max(jnp.where(valid, e, jnp.float32(-1e30)), axis=1, keepdims=True)
    ex = jnp.where(valid, jnp.exp(e - m), jnp.float32(0.0))
    ki = lax.broadcasted_iota(jnp.int32, (_T, _T), 0)
    ji = lax.broadcasted_iota(jnp.int32, (_T, _T), 1)
    tri = jnp.where(ki >= ji + 1, jnp.float32(1.0), jnp.float32(0.0))
    suf = jnp.dot(ex, tri, preferred_element_type=jnp.float32)
    total = (jnp.sum(jnp.log(suf[:, : _T - 1]))
             + jnp.float32(_T - 1) * jnp.sum(m)
             - jnp.sum(e[:, 1:]))
    out_ref[0, 0] = total / jnp.float32(_B * (_T - 1))


def _loss_call(x):
    out = pl.pallas_call(
        _loss_body,
        out_shape=jax.ShapeDtypeStruct((1, 1), jnp.float32),
        in_specs=[pl.BlockSpec((_B, 2 * _IDX_W * _LANES), lambda: (0, 0))],
        out_specs=pl.BlockSpec(memory_space=pltpu.SMEM),
    )(x)
    return out[0, 0]


def kernel(encoder_output, his_turn_end_ids, W_ih, W_hh, b_ih, b_hh, con_W, con_b):
    table = encoder_output.reshape(_B * _SEQ, _ENC)
    ids_flat = his_turn_end_ids.astype(jnp.int32).reshape(_PAIRS)
    x = _sc_gather_dot()(table, ids_flat, con_W)
    return _loss_call(x)


# speedup vs baseline: 1.0019x; 1.0019x over previous
import functools

import jax
import jax.numpy as jnp
from jax import lax
from jax.experimental import pallas as pl
from jax.experimental.pallas import tpu as pltpu
from jax.experimental.pallas import tpu_sc as plsc

_B = 16
_SEQ = 2048
_ENC = 1024
_T = 50

_NC = 2
_NS = 16
_NW = _NC * _NS
_LANES = 16
_PAIRS = _B * _T
_ROWS_PER_W = _PAIRS // _NW
_IDX_W = 32
_CHUNKS = _ENC // _LANES


def _sc_body(table_hbm, ids_hbm, conw_hbm, e_hbm, ids_v, idx_a, idx_b,
             rows_a, rows_b, w_v, e_v, sem, sem_b, sem_w):
    wid = lax.axis_index("s") * _NC + lax.axis_index("c")
    b = wid // 2
    half = wid % 2
    cp_w = pltpu.async_copy(conw_hbm.at[0, pl.ds(_ENC, _ENC)], w_v, sem_w)
    base = wid * _ROWS_PER_W
    abase = (base // 8) * 8
    sh = base - abase
    pltpu.sync_copy(ids_hbm.at[pl.ds(abase, 32)], ids_v.at[pl.ds(0, 32)])
    lane = lax.iota(jnp.int32, _LANES)
    off = b * _SEQ
    nb = _ROWS_PER_W - _LANES
    idx_a[pl.ds(0, _LANES)] = ids_v[pl.ds(sh, _LANES)] + off
    tail = ids_v[pl.ds(sh + _LANES, _LANES)] + off
    npad = _IDX_W - _ROWS_PER_W
    pad_vals = wid * npad + (lane - nb)
    idx_b[pl.ds(0, _LANES)] = jnp.where(lane < nb, tail, pad_vals)
    cp_a = pltpu.async_copy(table_hbm.at[idx_a], rows_a, sem)
    cp_b = pltpu.async_copy(table_hbm.at[idx_b], rows_b, sem_b)
    cp_w.wait()

    def make_mac(rows_ref, nrows):
        def mac(l, accs):
            off = l * _LANES
            w_c = w_v[pl.ds(off, _LANES)]
            return tuple(accs[r] + rows_ref[r, pl.ds(off, _LANES)] * w_c
                         for r in range(nrows))
        return mac
    zero = jnp.zeros((_LANES,), jnp.float32)
    cp_a.wait()
    accs_a = lax.fori_loop(0, _CHUNKS, make_mac(rows_a, _LANES),
                           (zero,) * _LANES)
    cp_b.wait()
    accs_b = lax.fori_loop(0, _CHUNKS, make_mac(rows_b, nb), (zero,) * nb)
    for r, acc in enumerate(accs_a + accs_b):
        e_v[pl.ds(r * _LANES, _LANES)] = acc
    for r in range(_ROWS_PER_W, _IDX_W):
        e_v[pl.ds(r * _LANES, _LANES)] = zero
    pltpu.sync_copy(e_v, e_hbm.at[b, pl.ds(half * (_IDX_W * _LANES),
                                           _IDX_W * _LANES)])


@functools.lru_cache(maxsize=1)
def _sc_gather_dot():
    return functools.partial(
        pl.kernel,
        mesh=plsc.VectorSubcoreMesh(core_axis_name="c", subcore_axis_name="s"),
        out_type=jax.ShapeDtypeStruct((_B, 2 * _IDX_W * _LANES), jnp.float32),
        scratch_types=[
            pltpu.VMEM((48,), jnp.int32),
            pltpu.VMEM((_LANES,), jnp.int32),
            pltpu.VMEM((_LANES,), jnp.int32),
            pltpu.VMEM((_LANES, _ENC), jnp.float32),
            pltpu.VMEM((_LANES, _ENC), jnp.float32),
            pltpu.VMEM((_ENC,), jnp.float32),
            pltpu.VMEM((_IDX_W * _LANES,), jnp.float32),
            pltpu.SemaphoreType.DMA,
            pltpu.SemaphoreType.DMA,
            pltpu.SemaphoreType.DMA,
        ],
    )(_sc_body)


def _loss_body(x_ref, out_ref):
    x = x_ref[:, :]
    _W = _IDX_W * _LANES
    ii = lax.broadcasted_iota(jnp.int32, (2 * _W, _T), 0)
    tt = lax.broadcasted_iota(jnp.int32, (2 * _W, _T), 1)
    slot = (ii % _W) // _LANES
    half = ii // _W
    fold = jnp.where((slot < _ROWS_PER_W) & (half * _ROWS_PER_W + slot == tt),
                     jnp.float32(1.0), jnp.float32(0.0))
    e = jnp.dot(x, fold, preferred_element_type=jnp.float32)
    kcol = lax.broadcasted_iota(jnp.int32, (_B, _T), 1)
    valid = kcol >= 1
    m = jnp.max(jnp.where(valid, e, jnp.float32(-1e30)), axis=1, keepdims=True)
    ex = jnp.where(valid, jnp.exp(e - m), jnp.float32(0.0))
    ki = lax.broadcasted_iota(jnp.int32, (_T, _T), 0)
    ji = lax.broadcasted_iota(jnp.int32, (_T, _T), 1)
    tri = jnp.where(ki >= ji + 1, jnp.float32(1.0), jnp.float32(0.0))
    suf = jnp.dot(ex, tri, preferred_element_type=jnp.float32)
    total = (jnp.sum(jnp.log(suf[:, : _T - 1]))
             + jnp.float32(_T - 1) * jnp.sum(m)
             - jnp.sum(e[:, 1:]))
    out_ref[0, 0] = total / jnp.float32(_B * (_T - 1))


def _loss_call(x):
    out = pl.pallas_call(
        _loss_body,
        out_shape=jax.ShapeDtypeStruct((1, 1), jnp.float32),
        in_specs=[pl.BlockSpec((_B, 2 * _IDX_W * _LANES), lambda: (0, 0))],
        out_specs=pl.BlockSpec(memory_space=pltpu.SMEM),
    )(x)
    return out[0, 0]


def kernel(encoder_output, his_turn_end_ids, W_ih, W_hh, b_ih, b_hh, con_W, con_b):
    table = encoder_output.reshape(_B * _SEQ, _ENC)
    ids_flat = his_turn_end_ids.astype(jnp.int32).reshape(_PAIRS)
    x = _sc_gather_dot()(table, ids_flat, con_W)
    return _loss_call(x)
